# Initial kernel scaffold; baseline (speedup 1.0000x reference)
import jax
import jax.numpy as jnp
from jax.experimental import pallas as pl


def kernel(h, pos, edge_index, edge_attr, We1, be1, We2, be2, Wn1, bn1, Wn2, bn2, Wc1, bc1, Wc2):
    raise NotImplementedError("write your pallas kernel here")


# speedup vs baseline: 3.8735x; 3.8735x over previous
import functools

import jax
import jax.numpy as jnp
from jax import lax
from jax.experimental import pallas as pl
from jax.experimental.pallas import tpu as pltpu
from jax.experimental.pallas import tpu_sc as plsc

N = 10000
E = 320000
HID = 128
EDGE_DIM = 16
PW = 16

NC = 2
NS = 16
NW = NC * NS
EW = E // NW
C = 80
NCH = EW // C
RPT = N // NS

_f32 = jnp.float32


@functools.lru_cache(maxsize=None)
def _sc_mesh():
    return plsc.VectorSubcoreMesh(core_axis_name="c", subcore_axis_name="s",
                                  num_cores=NC, num_subcores=NS)


def _gather_body(row_h, col_h, a_h, b_h, p_h,
                 s_out_h, rel_h,
                 idxr, idxc,
                 ba0, bb0, bpr0, bpc0,
                 ba1, bb1, bpr1, bpc1,
                 sg0, sg1, sw0, sw1):
    cid = lax.axis_index("c")
    sid = lax.axis_index("s")
    wid = sid * NC + cid

    pltpu.sync_copy(row_h.at[wid], idxr)
    pltpu.sync_copy(col_h.at[wid], idxc)

    def startg(j, ba, bb, bpr, bpc, sg):
        ia = idxr.at[j]
        ic = idxc.at[j]
        pltpu.async_copy(a_h.at[ia], ba, sg)
        pltpu.async_copy(b_h.at[ic], bb, sg)
        pltpu.async_copy(p_h.at[ia], bpr, sg)
        pltpu.async_copy(p_h.at[ic], bpc, sg)

    def waitg(ba, bb, bpr, bpc, sg):
        pltpu.make_async_copy(a_h.at[pl.ds(0, C)], ba, sg).wait()
        pltpu.make_async_copy(b_h.at[pl.ds(0, C)], bb, sg).wait()
        pltpu.make_async_copy(p_h.at[pl.ds(0, C)], bpr, sg).wait()
        pltpu.make_async_copy(p_h.at[pl.ds(0, C)], bpc, sg).wait()

    def combine(ba, bb, bpr, bpc):
        def vrow(i, c2):
            for k in range(HID // 16):
                sl = (i, pl.ds(k * 16, 16))
                ba[sl] = ba[sl] + bb[sl]
            pp = (i, pl.ds(0, 16))
            bpr[pp] = bpr[pp] - bpc[pp]
            return c2
        lax.fori_loop(0, C, vrow, 0)

    def startw(j, ba, bpr, sw):
        base = wid * EW + j * C
        pltpu.async_copy(ba, s_out_h.at[pl.ds(base, C)], sw)
        pltpu.async_copy(bpr, rel_h.at[pl.ds(base, C)], sw)

    def waitw(ba, bpr, sw):
        pltpu.make_async_copy(ba, s_out_h.at[pl.ds(0, C)], sw).wait()
        pltpu.make_async_copy(bpr, rel_h.at[pl.ds(0, C)], sw).wait()

    startg(0, ba0, bb0, bpr0, bpc0, sg0)

    def pipe(jj, carry):
        j0 = 2 * jj
        j1 = j0 + 1
        startg(j1, ba1, bb1, bpr1, bpc1, sg1)
        waitg(ba0, bb0, bpr0, bpc0, sg0)
        combine(ba0, bb0, bpr0, bpc0)
        startw(j0, ba0, bpr0, sw0)
        waitg(ba1, bb1, bpr1, bpc1, sg1)
        combine(ba1, bb1, bpr1, bpc1)
        startw(j1, ba1, bpr1, sw1)
        waitw(ba0, bpr0, sw0)
        startg(j0 + 2, ba0, bb0, bpr0, bpc0, sg0)
        waitw(ba1, bpr1, sw1)
        return carry

    lax.fori_loop(0, (NCH - 1) // 2, pipe, 0)
    waitg(ba0, bb0, bpr0, bpc0, sg0)
    combine(ba0, bb0, bpr0, bpc0)
    startw(NCH - 1, ba0, bpr0, sw0)
    waitw(ba0, bpr0, sw0)


@functools.lru_cache(maxsize=None)
def _gather():
  return pl.kernel(
    _gather_body,
    out_type=(
        jax.ShapeDtypeStruct((E, HID), _f32),
        jax.ShapeDtypeStruct((E, PW), _f32),
    ),
    mesh=_sc_mesh(),
    compiler_params=pltpu.CompilerParams(use_tc_tiling_on_sc=False),
    scratch_types=[
        pltpu.VMEM((NCH, C), jnp.int32),
        pltpu.VMEM((NCH, C), jnp.int32),
        pltpu.VMEM((C, HID), _f32),
        pltpu.VMEM((C, HID), _f32),
        pltpu.VMEM((C, PW), _f32),
        pltpu.VMEM((C, PW), _f32),
        pltpu.VMEM((C, HID), _f32),
        pltpu.VMEM((C, HID), _f32),
        pltpu.VMEM((C, PW), _f32),
        pltpu.VMEM((C, PW), _f32),
        pltpu.SemaphoreType.DMA,
        pltpu.SemaphoreType.DMA,
        pltpu.SemaphoreType.DMA,
        pltpu.SemaphoreType.DMA,
    ],
  )


def _scatter_body(col_h, m_h, c_h, z128_h, z16_h,
                  outm_h, outc_h,
                  idxc, bm0, bc0, bm1, bc1, m_acc, c_acc,
                  sl0, sl1):
    cid = lax.axis_index("c")
    sid = lax.axis_index("s")
    wid = sid * NC + cid

    rows = pl.ds(sid * RPT, RPT)
    pltpu.sync_copy(z128_h.at[rows], m_acc.at[rows])
    pltpu.sync_copy(z16_h.at[rows], c_acc.at[rows])
    pltpu.sync_copy(col_h.at[wid], idxc)
    plsc.subcore_barrier()

    def startl(j, bm, bc, sl):
        base = wid * EW + j * C
        pltpu.async_copy(m_h.at[pl.ds(base, C)], bm, sl)
        pltpu.async_copy(c_h.at[pl.ds(base, C)], bc, sl)

    def waitl(bm, bc, sl):
        pltpu.make_async_copy(m_h.at[pl.ds(0, C)], bm, sl).wait()
        pltpu.make_async_copy(c_h.at[pl.ds(0, C)], bc, sl).wait()

    def scat(j, bm, bc):
        ic = idxc.at[j]
        pltpu.sync_copy(bm, m_acc.at[ic], add=True)
        pltpu.sync_copy(bc, c_acc.at[ic], add=True)

    startl(0, bm0, bc0, sl0)

    def pipe(jj, carry):
        j0 = 2 * jj
        j1 = j0 + 1
        startl(j1, bm1, bc1, sl1)
        waitl(bm0, bc0, sl0)
        scat(j0, bm0, bc0)
        startl(j0 + 2, bm0, bc0, sl0)
        waitl(bm1, bc1, sl1)
        scat(j1, bm1, bc1)
        return carry

    lax.fori_loop(0, (NCH - 1) // 2, pipe, 0)
    waitl(bm0, bc0, sl0)
    scat(NCH - 1, bm0, bc0)

    plsc.subcore_barrier()
    pltpu.sync_copy(m_acc.at[rows], outm_h.at[cid, rows])
    pltpu.sync_copy(c_acc.at[rows], outc_h.at[cid, rows])


@functools.lru_cache(maxsize=None)
def _scatter():
  return pl.kernel(
    _scatter_body,
    out_type=(
        jax.ShapeDtypeStruct((NC, N, HID), _f32),
        jax.ShapeDtypeStruct((NC, N, PW), _f32),
    ),
    mesh=_sc_mesh(),
    compiler_params=pltpu.CompilerParams(use_tc_tiling_on_sc=False),
    scratch_types=[
        pltpu.VMEM((NCH, C), jnp.int32),
        pltpu.VMEM((C, HID), _f32),
        pltpu.VMEM((C, PW), _f32),
        pltpu.VMEM((C, HID), _f32),
        pltpu.VMEM((C, PW), _f32),
        pltpu.VMEM_SHARED((N, HID), _f32),
        pltpu.VMEM_SHARED((N, PW), _f32),
        pltpu.SemaphoreType.DMA,
        pltpu.SemaphoreType.DMA,
    ],
  )


def _prep_body(h_ref, wa_ref, wb_ref, be1_ref, a_ref, b_ref):
    hv = h_ref[...]
    a_ref[...] = (jnp.dot(hv, wa_ref[...], preferred_element_type=_f32)
                  + be1_ref[...])
    b_ref[...] = jnp.dot(hv, wb_ref[...], preferred_element_type=_f32)


def _edge_body(s_ref, rel_ref, ea_ref,
               wc_ref, wd_ref, we2_ref, be2_ref, wc1_ref, bc1_ref, wc2_ref,
               m_ref, co_ref):
    rel = rel_ref[:, :3]
    dist = jnp.sqrt(jnp.sum(rel * rel, axis=1, keepdims=True))
    t = (s_ref[...]
         + jnp.dot(ea_ref[...], wc_ref[...], preferred_element_type=_f32)
         + dist * wd_ref[...])
    t = jax.nn.silu(t)
    m = jax.nn.silu(jnp.dot(t, we2_ref[...], preferred_element_type=_f32)
                    + be2_ref[...])
    m_ref[...] = m
    cw = jnp.dot(
        jax.nn.silu(jnp.dot(m, wc1_ref[...], preferred_element_type=_f32)
                    + bc1_ref[...]),
        wc2_ref[...], preferred_element_type=_f32)
    co = cw * rel / (dist + 1e-8)
    co_ref[...] = jnp.concatenate(
        [co, jnp.zeros((co.shape[0], PW - 3), co.dtype)], axis=1)


def _node_body(h_ref, m0_ref, m1_ref, p0_ref, p1_ref, pos_ref,
               wn1a_ref, wn1b_ref, bn1_ref, wn2_ref, bn2_ref,
               hn_ref, pn_ref):
    hv = h_ref[...]
    mi = m0_ref[...] + m1_ref[...]
    t = jax.nn.silu(jnp.dot(hv, wn1a_ref[...], preferred_element_type=_f32)
                    + jnp.dot(mi, wn1b_ref[...], preferred_element_type=_f32)
                    + bn1_ref[...])
    hn_ref[...] = hv + jnp.dot(t, wn2_ref[...], preferred_element_type=_f32) \
        + bn2_ref[...]
    pn_ref[...] = pos_ref[...] + (p0_ref[...] + p1_ref[...])[:, :3]


def _row_block(blk, width):
    return pl.BlockSpec((blk, width), lambda i: (i, 0))


def _full(shape):
    return pl.BlockSpec(shape, lambda i: tuple(0 for _ in shape))


NBLK = 2000
EBLK = 2560


def _prep(h, wa, wb, be1):
    return pl.pallas_call(
        _prep_body,
        grid=(N // NBLK,),
        in_specs=[
            _row_block(NBLK, HID),
            _full((HID, HID)),
            _full((HID, HID)),
            _full((1, HID)),
        ],
        out_specs=[_row_block(NBLK, HID), _row_block(NBLK, HID)],
        out_shape=[
            jax.ShapeDtypeStruct((N, HID), _f32),
            jax.ShapeDtypeStruct((N, HID), _f32),
        ],
    )(h, wa, wb, be1)


def _edge(s, rel, ea, wc, wd, we2, be2, wc1, bc1, wc2):
    return pl.pallas_call(
        _edge_body,
        grid=(E // EBLK,),
        in_specs=[
            _row_block(EBLK, HID),
            _row_block(EBLK, PW),
            _row_block(EBLK, EDGE_DIM),
            _full((EDGE_DIM, HID)),
            _full((1, HID)),
            _full((HID, HID)),
            _full((1, HID)),
            _full((HID, HID)),
            _full((1, HID)),
            _full((HID, 1)),
        ],
        out_specs=[_row_block(EBLK, HID), _row_block(EBLK, PW)],
        out_shape=[
            jax.ShapeDtypeStruct((E, HID), _f32),
            jax.ShapeDtypeStruct((E, PW), _f32),
        ],
        name="edge_mlp",
    )(s, rel, ea, wc, wd, we2, be2, wc1, bc1, wc2)


def _node(h, m0, m1, p0, p1, pos, wn1a, wn1b, bn1, wn2, bn2):
    return pl.pallas_call(
        _node_body,
        grid=(N // NBLK,),
        in_specs=[
            _row_block(NBLK, HID),
            _row_block(NBLK, HID),
            _row_block(NBLK, HID),
            _row_block(NBLK, PW),
            _row_block(NBLK, PW),
            _row_block(NBLK, 3),
            _full((HID, HID)),
            _full((HID, HID)),
            _full((1, HID)),
            _full((HID, HID)),
            _full((1, HID)),
        ],
        out_specs=[_row_block(NBLK, HID), _row_block(NBLK, 3)],
        out_shape=[
            jax.ShapeDtypeStruct((N, HID), _f32),
            jax.ShapeDtypeStruct((N, 3), _f32),
        ],
    )(h, m0, m1, p0, p1, pos, wn1a, wn1b, bn1, wn2, bn2)


def kernel(h, pos, edge_index, edge_attr,
           We1, be1, We2, be2, Wn1, bn1, Wn2, bn2, Wc1, bc1, Wc2):
    row = edge_index[0].astype(jnp.int32).reshape(NW, NCH, C)
    col = edge_index[1].astype(jnp.int32)
    col3 = col.reshape(NW, NCH, C)
    pos16 = jnp.pad(pos, ((0, 0), (0, PW - 3)))

    a_tab, b_tab = _prep(h, We1[:HID], We1[HID:2 * HID],
                         be1.reshape(1, HID))
    s_arr, rel = _gather()(row, col3, a_tab, b_tab, pos16)

    wc = We1[2 * HID:2 * HID + EDGE_DIM]
    wd = We1[2 * HID + EDGE_DIM:].reshape(1, HID)
    m_ij, co = _edge(s_arr, rel, edge_attr, wc, wd,
                     We2, be2.reshape(1, HID),
                     Wc1, bc1.reshape(1, HID), Wc2)

    z128 = jnp.zeros((N, HID), _f32)
    z16 = jnp.zeros((N, PW), _f32)
    m_parts, c_parts = _scatter()(col3, m_ij, co, z128, z16)

    h_new, pos_new = _node(h, m_parts[0], m_parts[1],
                           c_parts[0], c_parts[1], pos,
                           Wn1[:HID], Wn1[HID:], bn1.reshape(1, HID),
                           Wn2, bn2.reshape(1, HID))
    return h_new, pos_new


# speedup vs baseline: 4.3824x; 1.1314x over previous
import functools

import jax
import jax.numpy as jnp
from jax import lax
from jax.experimental import pallas as pl
from jax.experimental.pallas import tpu as pltpu
from jax.experimental.pallas import tpu_sc as plsc

N = 10000
E = 320000
HID = 128
EDGE_DIM = 16
PW = 16

NC = 2
NS = 16
NW = NC * NS
EW = E // NW
C = 80
NCH = EW // C
RPT = N // NS

_f32 = jnp.float32


@functools.lru_cache(maxsize=None)
def _sc_mesh():
    return plsc.VectorSubcoreMesh(core_axis_name="c", subcore_axis_name="s",
                                  num_cores=NC, num_subcores=NS)


def _make_gather_body(nch):
  ewh = nch * C

  def _gather_body(row_h, col_h, a_h, b_h, p_h,
                   s_out_h, rel_h,
                   idxr, idxc,
                   ba0, bb0, bpr0, bpc0,
                   ba1, bb1, bpr1, bpc1,
                   sg0, sg1, sw0, sw1):
    cid = lax.axis_index("c")
    sid = lax.axis_index("s")
    wid = sid * NC + cid

    pltpu.sync_copy(row_h.at[wid], idxr)
    pltpu.sync_copy(col_h.at[wid], idxc)

    def startg(j, ba, bb, bpr, bpc, sg):
        ia = idxr.at[j]
        ic = idxc.at[j]
        pltpu.async_copy(a_h.at[ia], ba, sg)
        pltpu.async_copy(b_h.at[ic], bb, sg)
        pltpu.async_copy(p_h.at[ia], bpr, sg)
        pltpu.async_copy(p_h.at[ic], bpc, sg)

    def waitg(ba, bb, bpr, bpc, sg):
        pltpu.make_async_copy(a_h.at[pl.ds(0, C)], ba, sg).wait()
        pltpu.make_async_copy(b_h.at[pl.ds(0, C)], bb, sg).wait()
        pltpu.make_async_copy(p_h.at[pl.ds(0, C)], bpr, sg).wait()
        pltpu.make_async_copy(p_h.at[pl.ds(0, C)], bpc, sg).wait()

    def combine(ba, bb, bpr, bpc):
        def vrow(i, c2):
            for k in range(HID // 16):
                sl = (i, pl.ds(k * 16, 16))
                ba[sl] = ba[sl] + bb[sl]
            pp = (i, pl.ds(0, 16))
            bpr[pp] = bpr[pp] - bpc[pp]
            return c2
        lax.fori_loop(0, C, vrow, 0)

    def startw(j, ba, bpr, sw):
        base = wid * ewh + j * C
        pltpu.async_copy(ba, s_out_h.at[pl.ds(base, C)], sw)
        pltpu.async_copy(bpr, rel_h.at[pl.ds(base, C)], sw)

    def waitw(ba, bpr, sw):
        pltpu.make_async_copy(ba, s_out_h.at[pl.ds(0, C)], sw).wait()
        pltpu.make_async_copy(bpr, rel_h.at[pl.ds(0, C)], sw).wait()

    set0 = (ba0, bb0, bpr0, bpc0)
    set1 = (ba1, bb1, bpr1, bpc1)
    startg(0, *set0, sg0)

    def pipe(jj, carry):
        j0 = 2 * jj
        j1 = j0 + 1
        startg(j1, *set1, sg1)
        waitg(*set0, sg0)
        combine(*set0)
        startw(j0, ba0, bpr0, sw0)
        waitg(*set1, sg1)
        combine(*set1)
        startw(j1, ba1, bpr1, sw1)
        waitw(ba0, bpr0, sw0)
        startg(j0 + 2, *set0, sg0)
        waitw(ba1, bpr1, sw1)
        return carry

    lax.fori_loop(0, (nch - 1) // 2, pipe, 0)
    if nch % 2 == 1:
        waitg(*set0, sg0)
        combine(*set0)
        startw(nch - 1, ba0, bpr0, sw0)
        waitw(ba0, bpr0, sw0)
    else:
        startg(nch - 1, *set1, sg1)
        waitg(*set0, sg0)
        combine(*set0)
        startw(nch - 2, ba0, bpr0, sw0)
        waitg(*set1, sg1)
        combine(*set1)
        startw(nch - 1, ba1, bpr1, sw1)
        waitw(ba0, bpr0, sw0)
        waitw(ba1, bpr1, sw1)

  return _gather_body


@functools.lru_cache(maxsize=None)
def _gather(nch):
  eh = NW * nch * C
  return pl.kernel(
    _make_gather_body(nch),
    out_type=(
        jax.ShapeDtypeStruct((eh, HID), _f32),
        jax.ShapeDtypeStruct((eh, PW), _f32),
    ),
    mesh=_sc_mesh(),
    compiler_params=pltpu.CompilerParams(use_tc_tiling_on_sc=False),
    scratch_types=[
        pltpu.VMEM((nch, C), jnp.int32),
        pltpu.VMEM((nch, C), jnp.int32),
        pltpu.VMEM((C, HID), _f32),
        pltpu.VMEM((C, HID), _f32),
        pltpu.VMEM((C, PW), _f32),
        pltpu.VMEM((C, PW), _f32),
        pltpu.VMEM((C, HID), _f32),
        pltpu.VMEM((C, HID), _f32),
        pltpu.VMEM((C, PW), _f32),
        pltpu.VMEM((C, PW), _f32),
        pltpu.SemaphoreType.DMA,
        pltpu.SemaphoreType.DMA,
        pltpu.SemaphoreType.DMA,
        pltpu.SemaphoreType.DMA,
    ],
  )


def _make_scatter_body(nch0, nch1):
  def _scatter_body(col_h, m0_h, c0_h, m1_h, c1_h, z128_h, z16_h,
                    outm_h, outc_h,
                    idxc, bm0, bc0, bm1, bc1, m_acc, c_acc,
                    sl0, sl1):
    cid = lax.axis_index("c")
    sid = lax.axis_index("s")
    wid = sid * NC + cid

    rows = pl.ds(sid * RPT, RPT)
    pltpu.sync_copy(z128_h.at[rows], m_acc.at[rows])
    pltpu.sync_copy(z16_h.at[rows], c_acc.at[rows])
    pltpu.sync_copy(col_h.at[wid], idxc)
    plsc.subcore_barrier()

    def run(m_h, c_h, joff, nch):
        ewh = nch * C

        def startl(j, bm, bc, sl):
            base = wid * ewh + j * C
            pltpu.async_copy(m_h.at[pl.ds(base, C)], bm, sl)
            pltpu.async_copy(c_h.at[pl.ds(base, C)], bc, sl)

        def waitl(bm, bc, sl):
            pltpu.make_async_copy(m_h.at[pl.ds(0, C)], bm, sl).wait()
            pltpu.make_async_copy(c_h.at[pl.ds(0, C)], bc, sl).wait()

        def scat(j, bm, bc):
            ic = idxc.at[joff + j]
            pltpu.sync_copy(bm, m_acc.at[ic], add=True)
            pltpu.sync_copy(bc, c_acc.at[ic], add=True)

        startl(0, bm0, bc0, sl0)

        def pipe(jj, carry):
            j0 = 2 * jj
            j1 = j0 + 1
            startl(j1, bm1, bc1, sl1)
            waitl(bm0, bc0, sl0)
            scat(j0, bm0, bc0)
            startl(j0 + 2, bm0, bc0, sl0)
            waitl(bm1, bc1, sl1)
            scat(j1, bm1, bc1)
            return carry

        lax.fori_loop(0, (nch - 1) // 2, pipe, 0)
        if nch % 2 == 1:
            waitl(bm0, bc0, sl0)
            scat(nch - 1, bm0, bc0)
        else:
            startl(nch - 1, bm1, bc1, sl1)
            waitl(bm0, bc0, sl0)
            scat(nch - 2, bm0, bc0)
            waitl(bm1, bc1, sl1)
            scat(nch - 1, bm1, bc1)

    run(m0_h, c0_h, 0, nch0)
    run(m1_h, c1_h, nch0, nch1)

    plsc.subcore_barrier()
    pltpu.sync_copy(m_acc.at[rows], outm_h.at[cid, rows])
    pltpu.sync_copy(c_acc.at[rows], outc_h.at[cid, rows])

  return _scatter_body


@functools.lru_cache(maxsize=None)
def _scatter(nch0, nch1):
  return pl.kernel(
    _make_scatter_body(nch0, nch1),
    out_type=(
        jax.ShapeDtypeStruct((NC, N, HID), _f32),
        jax.ShapeDtypeStruct((NC, N, PW), _f32),
    ),
    mesh=_sc_mesh(),
    compiler_params=pltpu.CompilerParams(use_tc_tiling_on_sc=False),
    scratch_types=[
        pltpu.VMEM((NCH, C), jnp.int32),
        pltpu.VMEM((C, HID), _f32),
        pltpu.VMEM((C, PW), _f32),
        pltpu.VMEM((C, HID), _f32),
        pltpu.VMEM((C, PW), _f32),
        pltpu.VMEM_SHARED((N, HID), _f32),
        pltpu.VMEM_SHARED((N, PW), _f32),
        pltpu.SemaphoreType.DMA,
        pltpu.SemaphoreType.DMA,
    ],
  )


def _prep_body(h_ref, wa_ref, wb_ref, be1_ref, a_ref, b_ref):
    hv = h_ref[...]
    a_ref[...] = (jnp.dot(hv, wa_ref[...], preferred_element_type=_f32)
                  + be1_ref[...])
    b_ref[...] = jnp.dot(hv, wb_ref[...], preferred_element_type=_f32)


def _edge_body(s_ref, rel_ref, ea_ref,
               wc_ref, wd_ref, we2_ref, be2_ref, wc1_ref, bc1_ref, wc2_ref,
               m_ref, co_ref):
    rel = rel_ref[:, :3]
    dist = jnp.sqrt(jnp.sum(rel * rel, axis=1, keepdims=True))
    t = (s_ref[...]
         + jnp.dot(ea_ref[...], wc_ref[...], preferred_element_type=_f32)
         + dist * wd_ref[...])
    t = jax.nn.silu(t)
    m = jax.nn.silu(jnp.dot(t, we2_ref[...], preferred_element_type=_f32)
                    + be2_ref[...])
    m_ref[...] = m
    cw = jnp.dot(
        jax.nn.silu(jnp.dot(m, wc1_ref[...], preferred_element_type=_f32)
                    + bc1_ref[...]),
        wc2_ref[...], preferred_element_type=_f32)
    co = cw * rel / (dist + 1e-8)
    co_ref[...] = jnp.concatenate(
        [co, jnp.zeros((co.shape[0], PW - 3), co.dtype)], axis=1)


def _node_body(h_ref, m0_ref, m1_ref, p0_ref, p1_ref, pos_ref,
               wn1a_ref, wn1b_ref, bn1_ref, wn2_ref, bn2_ref,
               hn_ref, pn_ref):
    hv = h_ref[...]
    mi = m0_ref[...] + m1_ref[...]
    t = jax.nn.silu(jnp.dot(hv, wn1a_ref[...], preferred_element_type=_f32)
                    + jnp.dot(mi, wn1b_ref[...], preferred_element_type=_f32)
                    + bn1_ref[...])
    hn_ref[...] = hv + jnp.dot(t, wn2_ref[...], preferred_element_type=_f32) \
        + bn2_ref[...]
    pn_ref[...] = pos_ref[...] + (p0_ref[...] + p1_ref[...])[:, :3]


def _row_block(blk, width):
    return pl.BlockSpec((blk, width), lambda i: (i, 0))


def _full(shape):
    return pl.BlockSpec(shape, lambda i: tuple(0 for _ in shape))


NBLK = 2000
EBLK = 2560


def _prep(h, wa, wb, be1):
    return pl.pallas_call(
        _prep_body,
        grid=(N // NBLK,),
        in_specs=[
            _row_block(NBLK, HID),
            _full((HID, HID)),
            _full((HID, HID)),
            _full((1, HID)),
        ],
        out_specs=[_row_block(NBLK, HID), _row_block(NBLK, HID)],
        out_shape=[
            jax.ShapeDtypeStruct((N, HID), _f32),
            jax.ShapeDtypeStruct((N, HID), _f32),
        ],
    )(h, wa, wb, be1)


def _edge(s, rel, ea, wc, wd, we2, be2, wc1, bc1, wc2):
    eh = s.shape[0]
    return pl.pallas_call(
        _edge_body,
        grid=(eh // EBLK,),
        in_specs=[
            _row_block(EBLK, HID),
            _row_block(EBLK, PW),
            _row_block(EBLK, EDGE_DIM),
            _full((EDGE_DIM, HID)),
            _full((1, HID)),
            _full((HID, HID)),
            _full((1, HID)),
            _full((HID, HID)),
            _full((1, HID)),
            _full((HID, 1)),
        ],
        out_specs=[_row_block(EBLK, HID), _row_block(EBLK, PW)],
        out_shape=[
            jax.ShapeDtypeStruct((eh, HID), _f32),
            jax.ShapeDtypeStruct((eh, PW), _f32),
        ],
        name="edge_mlp",
    )(s, rel, ea, wc, wd, we2, be2, wc1, bc1, wc2)


def _node(h, m0, m1, p0, p1, pos, wn1a, wn1b, bn1, wn2, bn2):
    return pl.pallas_call(
        _node_body,
        grid=(N // NBLK,),
        in_specs=[
            _row_block(NBLK, HID),
            _row_block(NBLK, HID),
            _row_block(NBLK, HID),
            _row_block(NBLK, PW),
            _row_block(NBLK, PW),
            _row_block(NBLK, 3),
            _full((HID, HID)),
            _full((HID, HID)),
            _full((1, HID)),
            _full((HID, HID)),
            _full((1, HID)),
        ],
        out_specs=[_row_block(NBLK, HID), _row_block(NBLK, 3)],
        out_shape=[
            jax.ShapeDtypeStruct((N, HID), _f32),
            jax.ShapeDtypeStruct((N, 3), _f32),
        ],
    )(h, m0, m1, p0, p1, pos, wn1a, wn1b, bn1, wn2, bn2)


NCH0 = 63
NCH1 = NCH - NCH0


def kernel(h, pos, edge_index, edge_attr,
           We1, be1, We2, be2, Wn1, bn1, Wn2, bn2, Wc1, bc1, Wc2):
    row3 = edge_index[0].astype(jnp.int32).reshape(NW, NCH, C)
    col3 = edge_index[1].astype(jnp.int32).reshape(NW, NCH, C)
    pos16 = jnp.pad(pos, ((0, 0), (0, PW - 3)))

    ea4 = edge_attr.reshape(NW, NCH, C, EDGE_DIM)
    ea0 = ea4[:, :NCH0].reshape(NW * NCH0 * C, EDGE_DIM)
    ea1 = ea4[:, NCH0:].reshape(NW * NCH1 * C, EDGE_DIM)

    a_tab, b_tab = _prep(h, We1[:HID], We1[HID:2 * HID],
                         be1.reshape(1, HID))

    wc = We1[2 * HID:2 * HID + EDGE_DIM]
    wd = We1[2 * HID + EDGE_DIM:].reshape(1, HID)
    be2r = be2.reshape(1, HID)
    bc1r = bc1.reshape(1, HID)

    s0, rel0 = _gather(NCH0)(row3[:, :NCH0], col3[:, :NCH0],
                             a_tab, b_tab, pos16)
    s1, rel1 = _gather(NCH1)(row3[:, NCH0:], col3[:, NCH0:],
                             a_tab, b_tab, pos16)
    m0, co0 = _edge(s0, rel0, ea0, wc, wd, We2, be2r, Wc1, bc1r, Wc2)
    m1, co1 = _edge(s1, rel1, ea1, wc, wd, We2, be2r, Wc1, bc1r, Wc2)

    z128 = jnp.zeros((N, HID), _f32)
    z16 = jnp.zeros((N, PW), _f32)
    m_parts, c_parts = _scatter(NCH0, NCH1)(col3, m0, co0, m1, co1,
                                            z128, z16)

    h_new, pos_new = _node(h, m_parts[0], m_parts[1],
                           c_parts[0], c_parts[1], pos,
                           Wn1[:HID], Wn1[HID:], bn1.reshape(1, HID),
                           Wn2, bn2.reshape(1, HID))
    return h_new, pos_new


# speedup vs baseline: 4.6197x; 1.0541x over previous
import functools

import jax
import jax.numpy as jnp
from jax import lax
from jax.experimental import pallas as pl
from jax.experimental.pallas import tpu as pltpu
from jax.experimental.pallas import tpu_sc as plsc

N = 10000
E = 320000
HID = 128
EDGE_DIM = 16
PW = 16

NC = 2
NS = 16
NW = NC * NS
EW = E // NW
C = 80
NCH = EW // C
RPT = N // NS

_f32 = jnp.float32


@functools.lru_cache(maxsize=None)
def _sc_mesh():
    return plsc.VectorSubcoreMesh(core_axis_name="c", subcore_axis_name="s",
                                  num_cores=NC, num_subcores=NS)


def _make_gather_body(nch):
  ewh = nch * C

  def _gather_body(row_h, col_h, a_h, b_h, p_h,
                   s_out_h, rel_h,
                   idxr, idxc,
                   ba0, bb0, bpr0, bpc0,
                   ba1, bb1, bpr1, bpc1,
                   sg0, sg1, sw0, sw1):
    cid = lax.axis_index("c")
    sid = lax.axis_index("s")
    wid = sid * NC + cid

    pltpu.sync_copy(row_h.at[wid], idxr)
    pltpu.sync_copy(col_h.at[wid], idxc)

    def startg(j, ba, bb, bpr, bpc, sg):
        ia = idxr.at[j]
        ic = idxc.at[j]
        pltpu.async_copy(a_h.at[ia], ba, sg)
        pltpu.async_copy(b_h.at[ic], bb, sg)
        pltpu.async_copy(p_h.at[ia], bpr, sg)
        pltpu.async_copy(p_h.at[ic], bpc, sg)

    def waitg(ba, bb, bpr, bpc, sg):
        pltpu.make_async_copy(a_h.at[pl.ds(0, C)], ba, sg).wait()
        pltpu.make_async_copy(b_h.at[pl.ds(0, C)], bb, sg).wait()
        pltpu.make_async_copy(p_h.at[pl.ds(0, C)], bpr, sg).wait()
        pltpu.make_async_copy(p_h.at[pl.ds(0, C)], bpc, sg).wait()

    def combine(ba, bb, bpr, bpc):
        def vrow(i, c2):
            for k in range(HID // 16):
                sl = (i, pl.ds(k * 16, 16))
                ba[sl] = ba[sl] + bb[sl]
            pp = (i, pl.ds(0, 16))
            bpr[pp] = bpr[pp] - bpc[pp]
            return c2
        lax.fori_loop(0, C, vrow, 0)

    def startw(j, ba, bpr, sw):
        base = wid * ewh + j * C
        pltpu.async_copy(ba, s_out_h.at[pl.ds(base, C)], sw)
        pltpu.async_copy(bpr, rel_h.at[pl.ds(base, C)], sw)

    def waitw(ba, bpr, sw):
        pltpu.make_async_copy(ba, s_out_h.at[pl.ds(0, C)], sw).wait()
        pltpu.make_async_copy(bpr, rel_h.at[pl.ds(0, C)], sw).wait()

    set0 = (ba0, bb0, bpr0, bpc0)
    set1 = (ba1, bb1, bpr1, bpc1)
    startg(0, *set0, sg0)

    def pipe(jj, carry):
        j0 = 2 * jj
        j1 = j0 + 1
        startg(j1, *set1, sg1)
        waitg(*set0, sg0)
        combine(*set0)
        startw(j0, ba0, bpr0, sw0)
        waitg(*set1, sg1)
        combine(*set1)
        startw(j1, ba1, bpr1, sw1)
        waitw(ba0, bpr0, sw0)
        startg(j0 + 2, *set0, sg0)
        waitw(ba1, bpr1, sw1)
        return carry

    lax.fori_loop(0, (nch - 1) // 2, pipe, 0)
    if nch % 2 == 1:
        waitg(*set0, sg0)
        combine(*set0)
        startw(nch - 1, ba0, bpr0, sw0)
        waitw(ba0, bpr0, sw0)
    else:
        startg(nch - 1, *set1, sg1)
        waitg(*set0, sg0)
        combine(*set0)
        startw(nch - 2, ba0, bpr0, sw0)
        waitg(*set1, sg1)
        combine(*set1)
        startw(nch - 1, ba1, bpr1, sw1)
        waitw(ba0, bpr0, sw0)
        waitw(ba1, bpr1, sw1)

  return _gather_body


@functools.lru_cache(maxsize=None)
def _gather(nch):
  eh = NW * nch * C
  return pl.kernel(
    _make_gather_body(nch),
    out_type=(
        jax.ShapeDtypeStruct((eh, HID), _f32),
        jax.ShapeDtypeStruct((eh, PW), _f32),
    ),
    mesh=_sc_mesh(),
    compiler_params=pltpu.CompilerParams(use_tc_tiling_on_sc=False),
    scratch_types=[
        pltpu.VMEM((nch, C), jnp.int32),
        pltpu.VMEM((nch, C), jnp.int32),
        pltpu.VMEM((C, HID), _f32),
        pltpu.VMEM((C, HID), _f32),
        pltpu.VMEM((C, PW), _f32),
        pltpu.VMEM((C, PW), _f32),
        pltpu.VMEM((C, HID), _f32),
        pltpu.VMEM((C, HID), _f32),
        pltpu.VMEM((C, PW), _f32),
        pltpu.VMEM((C, PW), _f32),
        pltpu.SemaphoreType.DMA,
        pltpu.SemaphoreType.DMA,
        pltpu.SemaphoreType.DMA,
        pltpu.SemaphoreType.DMA,
    ],
  )


def _make_scatter_body(nch0, nch1):
  def _scatter_body(col0_h, col1_h, m0_h, c0_h, m1_h, c1_h, z128_h, z16_h,
                    outm_h, outc_h,
                    idxc, bm0, bc0, bm1, bc1, m_acc, c_acc,
                    sl0, sl1):
    cid = lax.axis_index("c")
    sid = lax.axis_index("s")
    wid = sid * NC + cid

    rows = pl.ds(sid * RPT, RPT)
    pltpu.sync_copy(z128_h.at[rows], m_acc.at[rows])
    pltpu.sync_copy(z16_h.at[rows], c_acc.at[rows])
    pltpu.sync_copy(col0_h.at[wid], idxc.at[pl.ds(0, nch0)])
    pltpu.sync_copy(col1_h.at[wid], idxc.at[pl.ds(nch0, nch1)])
    plsc.subcore_barrier()

    def run(m_h, c_h, joff, nch):
        ewh = nch * C

        def startl(j, bm, bc, sl):
            base = wid * ewh + j * C
            pltpu.async_copy(m_h.at[pl.ds(base, C)], bm, sl)
            pltpu.async_copy(c_h.at[pl.ds(base, C)], bc, sl)

        def waitl(bm, bc, sl):
            pltpu.make_async_copy(m_h.at[pl.ds(0, C)], bm, sl).wait()
            pltpu.make_async_copy(c_h.at[pl.ds(0, C)], bc, sl).wait()

        def scat(j, bm, bc):
            ic = idxc.at[joff + j]
            pltpu.sync_copy(bm, m_acc.at[ic], add=True)
            pltpu.sync_copy(bc, c_acc.at[ic], add=True)

        startl(0, bm0, bc0, sl0)

        def pipe(jj, carry):
            j0 = 2 * jj
            j1 = j0 + 1
            startl(j1, bm1, bc1, sl1)
            waitl(bm0, bc0, sl0)
            scat(j0, bm0, bc0)
            startl(j0 + 2, bm0, bc0, sl0)
            waitl(bm1, bc1, sl1)
            scat(j1, bm1, bc1)
            return carry

        lax.fori_loop(0, (nch - 1) // 2, pipe, 0)
        if nch % 2 == 1:
            waitl(bm0, bc0, sl0)
            scat(nch - 1, bm0, bc0)
        else:
            startl(nch - 1, bm1, bc1, sl1)
            waitl(bm0, bc0, sl0)
            scat(nch - 2, bm0, bc0)
            waitl(bm1, bc1, sl1)
            scat(nch - 1, bm1, bc1)

    run(m0_h, c0_h, 0, nch0)
    run(m1_h, c1_h, nch0, nch1)

    plsc.subcore_barrier()
    pltpu.sync_copy(m_acc.at[rows], outm_h.at[cid, rows])
    pltpu.sync_copy(c_acc.at[rows], outc_h.at[cid, rows])

  return _scatter_body


@functools.lru_cache(maxsize=None)
def _scatter(nch0, nch1):
  return pl.kernel(
    _make_scatter_body(nch0, nch1),
    out_type=(
        jax.ShapeDtypeStruct((NC, N, HID), _f32),
        jax.ShapeDtypeStruct((NC, N, PW), _f32),
    ),
    mesh=_sc_mesh(),
    compiler_params=pltpu.CompilerParams(use_tc_tiling_on_sc=False),
    scratch_types=[
        pltpu.VMEM((NCH, C), jnp.int32),
        pltpu.VMEM((C, HID), _f32),
        pltpu.VMEM((C, PW), _f32),
        pltpu.VMEM((C, HID), _f32),
        pltpu.VMEM((C, PW), _f32),
        pltpu.VMEM_SHARED((N, HID), _f32),
        pltpu.VMEM_SHARED((N, PW), _f32),
        pltpu.SemaphoreType.DMA,
        pltpu.SemaphoreType.DMA,
    ],
  )


def _prep_body(h_ref, wa_ref, wb_ref, be1_ref, a_ref, b_ref):
    hv = h_ref[...]
    a_ref[...] = (jnp.dot(hv, wa_ref[...], preferred_element_type=_f32)
                  + be1_ref[...])
    b_ref[...] = jnp.dot(hv, wb_ref[...], preferred_element_type=_f32)


def _edge_body(s_ref, rel_ref, ea_ref,
               wc_ref, wd_ref, we2_ref, be2_ref, wc1_ref, bc1_ref, wc2_ref,
               m_ref, co_ref):
    rel = rel_ref[:, :3]
    dist = jnp.sqrt(jnp.sum(rel * rel, axis=1, keepdims=True))
    t = (s_ref[...]
         + jnp.dot(ea_ref[...], wc_ref[...], preferred_element_type=_f32)
         + dist * wd_ref[...])
    t = jax.nn.silu(t)
    m = jax.nn.silu(jnp.dot(t, we2_ref[...], preferred_element_type=_f32)
                    + be2_ref[...])
    m_ref[...] = m
    cw = jnp.dot(
        jax.nn.silu(jnp.dot(m, wc1_ref[...], preferred_element_type=_f32)
                    + bc1_ref[...]),
        wc2_ref[...], preferred_element_type=_f32)
    co = cw * rel / (dist + 1e-8)
    co_ref[...] = jnp.concatenate(
        [co, jnp.zeros((co.shape[0], PW - 3), co.dtype)], axis=1)


def _node_body(h_ref, m0_ref, m1_ref, p0_ref, p1_ref, pos_ref,
               wn1a_ref, wn1b_ref, bn1_ref, wn2_ref, bn2_ref,
               hn_ref, pn_ref):
    hv = h_ref[...]
    mi = m0_ref[...] + m1_ref[...]
    t = jax.nn.silu(jnp.dot(hv, wn1a_ref[...], preferred_element_type=_f32)
                    + jnp.dot(mi, wn1b_ref[...], preferred_element_type=_f32)
                    + bn1_ref[...])
    hn_ref[...] = hv + jnp.dot(t, wn2_ref[...], preferred_element_type=_f32) \
        + bn2_ref[...]
    pn_ref[...] = pos_ref[...] + (p0_ref[...] + p1_ref[...])[:, :3]


def _row_block(blk, width):
    return pl.BlockSpec((blk, width), lambda i: (i, 0))


def _full(shape):
    return pl.BlockSpec(shape, lambda i: tuple(0 for _ in shape))


NBLK = 2000
EBLK = 2560


def _prep(h, wa, wb, be1):
    return pl.pallas_call(
        _prep_body,
        grid=(N // NBLK,),
        in_specs=[
            _row_block(NBLK, HID),
            _full((HID, HID)),
            _full((HID, HID)),
            _full((1, HID)),
        ],
        out_specs=[_row_block(NBLK, HID), _row_block(NBLK, HID)],
        out_shape=[
            jax.ShapeDtypeStruct((N, HID), _f32),
            jax.ShapeDtypeStruct((N, HID), _f32),
        ],
    )(h, wa, wb, be1)


def _edge(s, rel, ea, wc, wd, we2, be2, wc1, bc1, wc2):
    eh = s.shape[0]
    return pl.pallas_call(
        _edge_body,
        grid=(eh // EBLK,),
        in_specs=[
            _row_block(EBLK, HID),
            _row_block(EBLK, PW),
            _row_block(EBLK, EDGE_DIM),
            _full((EDGE_DIM, HID)),
            _full((1, HID)),
            _full((HID, HID)),
            _full((1, HID)),
            _full((HID, HID)),
            _full((1, HID)),
            _full((HID, 1)),
        ],
        out_specs=[_row_block(EBLK, HID), _row_block(EBLK, PW)],
        out_shape=[
            jax.ShapeDtypeStruct((eh, HID), _f32),
            jax.ShapeDtypeStruct((eh, PW), _f32),
        ],
        name="edge_mlp",
    )(s, rel, ea, wc, wd, we2, be2, wc1, bc1, wc2)


def _node(h, m0, m1, p0, p1, pos, wn1a, wn1b, bn1, wn2, bn2):
    return pl.pallas_call(
        _node_body,
        grid=(N // NBLK,),
        in_specs=[
            _row_block(NBLK, HID),
            _row_block(NBLK, HID),
            _row_block(NBLK, HID),
            _row_block(NBLK, PW),
            _row_block(NBLK, PW),
            _row_block(NBLK, 3),
            _full((HID, HID)),
            _full((HID, HID)),
            _full((1, HID)),
            _full((HID, HID)),
            _full((1, HID)),
        ],
        out_specs=[_row_block(NBLK, HID), _row_block(NBLK, 3)],
        out_shape=[
            jax.ShapeDtypeStruct((N, HID), _f32),
            jax.ShapeDtypeStruct((N, 3), _f32),
        ],
    )(h, m0, m1, p0, p1, pos, wn1a, wn1b, bn1, wn2, bn2)


NCH0 = 63
NCH1 = NCH - NCH0


E0 = NW * NCH0 * C
E1 = NW * NCH1 * C


def kernel(h, pos, edge_index, edge_attr,
           We1, be1, We2, be2, Wn1, bn1, Wn2, bn2, Wc1, bc1, Wc2):
    row = edge_index[0].astype(jnp.int32)
    col = edge_index[1].astype(jnp.int32)
    row0 = row[:E0].reshape(NW, NCH0, C)
    row1 = row[E0:].reshape(NW, NCH1, C)
    col0 = col[:E0].reshape(NW, NCH0, C)
    col1 = col[E0:].reshape(NW, NCH1, C)
    pos16 = jnp.pad(pos, ((0, 0), (0, PW - 3)))

    a_tab, b_tab = _prep(h, We1[:HID], We1[HID:2 * HID],
                         be1.reshape(1, HID))

    wc = We1[2 * HID:2 * HID + EDGE_DIM]
    wd = We1[2 * HID + EDGE_DIM:].reshape(1, HID)
    be2r = be2.reshape(1, HID)
    bc1r = bc1.reshape(1, HID)

    s0, rel0 = _gather(NCH0)(row0, col0, a_tab, b_tab, pos16)
    s1, rel1 = _gather(NCH1)(row1, col1, a_tab, b_tab, pos16)
    m0, co0 = _edge(s0, rel0, edge_attr[:E0], wc, wd,
                    We2, be2r, Wc1, bc1r, Wc2)
    m1, co1 = _edge(s1, rel1, edge_attr[E0:], wc, wd,
                    We2, be2r, Wc1, bc1r, Wc2)

    z128 = jnp.zeros((N, HID), _f32)
    z16 = jnp.zeros((N, PW), _f32)
    m_parts, c_parts = _scatter(NCH0, NCH1)(col0, col1, m0, co0, m1, co1,
                                            z128, z16)

    h_new, pos_new = _node(h, m_parts[0], m_parts[1],
                           c_parts[0], c_parts[1], pos,
                           Wn1[:HID], Wn1[HID:], bn1.reshape(1, HID),
                           Wn2, bn2.reshape(1, HID))
    return h_new, pos_new


# speedup vs baseline: 4.8428x; 1.0483x over previous
import functools

import jax
import jax.numpy as jnp
from jax import lax
from jax.experimental import pallas as pl
from jax.experimental.pallas import tpu as pltpu
from jax.experimental.pallas import tpu_sc as plsc

N = 10000
E = 320000
HID = 128
EDGE_DIM = 16
PW = 16

NC = 2
NS = 16
NW = NC * NS
EW = E // NW
C = 80
NCH = EW // C
RPT = N // NS

_f32 = jnp.float32


@functools.lru_cache(maxsize=None)
def _sc_mesh():
    return plsc.VectorSubcoreMesh(core_axis_name="c", subcore_axis_name="s",
                                  num_cores=NC, num_subcores=NS)


def _make_gather_body(nch):
  ewh = nch * C

  def _gather_body(row_h, col_h, a_h, b_h, p_h,
                   s_out_h, rel_h,
                   idxr, idxc,
                   ba0, bb0, bpr0, bpc0,
                   ba1, bb1, bpr1, bpc1,
                   sg0, sg1, sw0, sw1):
    cid = lax.axis_index("c")
    sid = lax.axis_index("s")
    wid = sid * NC + cid

    pltpu.sync_copy(row_h.at[wid], idxr)
    pltpu.sync_copy(col_h.at[wid], idxc)

    def startg(j, ba, bb, bpr, bpc, sg):
        ia = idxr.at[j]
        ic = idxc.at[j]
        pltpu.async_copy(a_h.at[ia], ba, sg)
        pltpu.async_copy(b_h.at[ic], bb, sg)
        pltpu.async_copy(p_h.at[ia], bpr, sg)
        pltpu.async_copy(p_h.at[ic], bpc, sg)

    def waitg(ba, bb, bpr, bpc, sg):
        pltpu.make_async_copy(a_h.at[pl.ds(0, C)], ba, sg).wait()
        pltpu.make_async_copy(b_h.at[pl.ds(0, C)], bb, sg).wait()
        pltpu.make_async_copy(p_h.at[pl.ds(0, C)], bpr, sg).wait()
        pltpu.make_async_copy(p_h.at[pl.ds(0, C)], bpc, sg).wait()

    def combine(ba, bb, bpr, bpc):
        def vrow(i, c2):
            for k in range(HID // 16):
                sl = (i, pl.ds(k * 16, 16))
                ba[sl] = ba[sl] + bb[sl]
            pp = (i, pl.ds(0, 16))
            bpr[pp] = bpr[pp] - bpc[pp]
            return c2
        lax.fori_loop(0, C, vrow, 0)

    def startw(j, ba, bpr, sw):
        base = wid * ewh + j * C
        pltpu.async_copy(ba, s_out_h.at[pl.ds(base, C)], sw)
        pltpu.async_copy(bpr, rel_h.at[pl.ds(base, C)], sw)

    def waitw(ba, bpr, sw):
        pltpu.make_async_copy(ba, s_out_h.at[pl.ds(0, C)], sw).wait()
        pltpu.make_async_copy(bpr, rel_h.at[pl.ds(0, C)], sw).wait()

    set0 = (ba0, bb0, bpr0, bpc0)
    set1 = (ba1, bb1, bpr1, bpc1)
    startg(0, *set0, sg0)

    def pipe(jj, carry):
        j0 = 2 * jj
        j1 = j0 + 1
        startg(j1, *set1, sg1)
        waitg(*set0, sg0)
        combine(*set0)
        startw(j0, ba0, bpr0, sw0)
        waitg(*set1, sg1)
        combine(*set1)
        startw(j1, ba1, bpr1, sw1)
        waitw(ba0, bpr0, sw0)
        startg(j0 + 2, *set0, sg0)
        waitw(ba1, bpr1, sw1)
        return carry

    lax.fori_loop(0, (nch - 1) // 2, pipe, 0)
    if nch % 2 == 1:
        waitg(*set0, sg0)
        combine(*set0)
        startw(nch - 1, ba0, bpr0, sw0)
        waitw(ba0, bpr0, sw0)
    else:
        startg(nch - 1, *set1, sg1)
        waitg(*set0, sg0)
        combine(*set0)
        startw(nch - 2, ba0, bpr0, sw0)
        waitg(*set1, sg1)
        combine(*set1)
        startw(nch - 1, ba1, bpr1, sw1)
        waitw(ba0, bpr0, sw0)
        waitw(ba1, bpr1, sw1)

  return _gather_body


@functools.lru_cache(maxsize=None)
def _gather(nch):
  eh = NW * nch * C
  return pl.kernel(
    _make_gather_body(nch),
    out_type=(
        jax.ShapeDtypeStruct((eh, HID), _f32),
        jax.ShapeDtypeStruct((eh, PW), _f32),
    ),
    mesh=_sc_mesh(),
    compiler_params=pltpu.CompilerParams(use_tc_tiling_on_sc=False),
    scratch_types=[
        pltpu.VMEM((nch, C), jnp.int32),
        pltpu.VMEM((nch, C), jnp.int32),
        pltpu.VMEM((C, HID), _f32),
        pltpu.VMEM((C, HID), _f32),
        pltpu.VMEM((C, PW), _f32),
        pltpu.VMEM((C, PW), _f32),
        pltpu.VMEM((C, HID), _f32),
        pltpu.VMEM((C, HID), _f32),
        pltpu.VMEM((C, PW), _f32),
        pltpu.VMEM((C, PW), _f32),
        pltpu.SemaphoreType.DMA,
        pltpu.SemaphoreType.DMA,
        pltpu.SemaphoreType.DMA,
        pltpu.SemaphoreType.DMA,
    ],
  )


def _make_scatter_body(nchs):
  np_ = len(nchs)

  def _scatter_body(*refs):
    col_hs = refs[0:np_]
    m_hs = refs[np_:2 * np_]
    c_hs = refs[2 * np_:3 * np_]
    z128_h, z16_h, outm_h, outc_h = refs[3 * np_:3 * np_ + 4]
    (idxc, bm0, bc0, bm1, bc1, m_acc, c_acc,
     sl0, sl1) = refs[3 * np_ + 4:]
    cid = lax.axis_index("c")
    sid = lax.axis_index("s")
    wid = sid * NC + cid

    rows = pl.ds(sid * RPT, RPT)
    pltpu.sync_copy(z128_h.at[rows], m_acc.at[rows])
    pltpu.sync_copy(z16_h.at[rows], c_acc.at[rows])
    off = 0
    for p in range(np_):
        pltpu.sync_copy(col_hs[p].at[wid], idxc.at[pl.ds(off, nchs[p])])
        off += nchs[p]
    plsc.subcore_barrier()

    def run(m_h, c_h, joff, nch):
        ewh = nch * C

        def startl(j, bm, bc, sl):
            base = wid * ewh + j * C
            pltpu.async_copy(m_h.at[pl.ds(base, C)], bm, sl)
            pltpu.async_copy(c_h.at[pl.ds(base, C)], bc, sl)

        def waitl(bm, bc, sl):
            pltpu.make_async_copy(m_h.at[pl.ds(0, C)], bm, sl).wait()
            pltpu.make_async_copy(c_h.at[pl.ds(0, C)], bc, sl).wait()

        def scat(j, bm, bc):
            ic = idxc.at[joff + j]
            pltpu.sync_copy(bm, m_acc.at[ic], add=True)
            pltpu.sync_copy(bc, c_acc.at[ic], add=True)

        startl(0, bm0, bc0, sl0)

        def pipe(jj, carry):
            j0 = 2 * jj
            j1 = j0 + 1
            startl(j1, bm1, bc1, sl1)
            waitl(bm0, bc0, sl0)
            scat(j0, bm0, bc0)
            startl(j0 + 2, bm0, bc0, sl0)
            waitl(bm1, bc1, sl1)
            scat(j1, bm1, bc1)
            return carry

        lax.fori_loop(0, (nch - 1) // 2, pipe, 0)
        if nch % 2 == 1:
            waitl(bm0, bc0, sl0)
            scat(nch - 1, bm0, bc0)
        else:
            startl(nch - 1, bm1, bc1, sl1)
            waitl(bm0, bc0, sl0)
            scat(nch - 2, bm0, bc0)
            waitl(bm1, bc1, sl1)
            scat(nch - 1, bm1, bc1)

    off = 0
    for p in range(np_):
        run(m_hs[p], c_hs[p], off, nchs[p])
        off += nchs[p]

    plsc.subcore_barrier()
    pltpu.sync_copy(m_acc.at[rows], outm_h.at[cid, rows])
    pltpu.sync_copy(c_acc.at[rows], outc_h.at[cid, rows])

  return _scatter_body


@functools.lru_cache(maxsize=None)
def _scatter(nchs):
  return pl.kernel(
    _make_scatter_body(nchs),
    out_type=(
        jax.ShapeDtypeStruct((NC, N, HID), _f32),
        jax.ShapeDtypeStruct((NC, N, PW), _f32),
    ),
    mesh=_sc_mesh(),
    compiler_params=pltpu.CompilerParams(use_tc_tiling_on_sc=False),
    scratch_types=[
        pltpu.VMEM((NCH, C), jnp.int32),
        pltpu.VMEM((C, HID), _f32),
        pltpu.VMEM((C, PW), _f32),
        pltpu.VMEM((C, HID), _f32),
        pltpu.VMEM((C, PW), _f32),
        pltpu.VMEM_SHARED((N, HID), _f32),
        pltpu.VMEM_SHARED((N, PW), _f32),
        pltpu.SemaphoreType.DMA,
        pltpu.SemaphoreType.DMA,
    ],
  )


def _prep_body(h_ref, wa_ref, wb_ref, be1_ref, a_ref, b_ref):
    hv = h_ref[...]
    a_ref[...] = (jnp.dot(hv, wa_ref[...], preferred_element_type=_f32)
                  + be1_ref[...])
    b_ref[...] = jnp.dot(hv, wb_ref[...], preferred_element_type=_f32)


def _edge_body(s_ref, rel_ref, ea_ref,
               wc_ref, wd_ref, we2_ref, be2_ref, wc1_ref, bc1_ref, wc2_ref,
               m_ref, co_ref):
    rel = rel_ref[:, :3]
    dist = jnp.sqrt(jnp.sum(rel * rel, axis=1, keepdims=True))
    t = (s_ref[...]
         + jnp.dot(ea_ref[...], wc_ref[...], preferred_element_type=_f32)
         + dist * wd_ref[...])
    t = jax.nn.silu(t)
    m = jax.nn.silu(jnp.dot(t, we2_ref[...], preferred_element_type=_f32)
                    + be2_ref[...])
    m_ref[...] = m
    cw = jnp.dot(
        jax.nn.silu(jnp.dot(m, wc1_ref[...], preferred_element_type=_f32)
                    + bc1_ref[...]),
        wc2_ref[...], preferred_element_type=_f32)
    co = cw * rel / (dist + 1e-8)
    co_ref[...] = jnp.concatenate(
        [co, jnp.zeros((co.shape[0], PW - 3), co.dtype)], axis=1)


def _node_body(h_ref, m0_ref, m1_ref, p0_ref, p1_ref, pos_ref,
               wn1a_ref, wn1b_ref, bn1_ref, wn2_ref, bn2_ref,
               hn_ref, pn_ref):
    hv = h_ref[...]
    mi = m0_ref[...] + m1_ref[...]
    t = jax.nn.silu(jnp.dot(hv, wn1a_ref[...], preferred_element_type=_f32)
                    + jnp.dot(mi, wn1b_ref[...], preferred_element_type=_f32)
                    + bn1_ref[...])
    hn_ref[...] = hv + jnp.dot(t, wn2_ref[...], preferred_element_type=_f32) \
        + bn2_ref[...]
    pn_ref[...] = pos_ref[...] + (p0_ref[...] + p1_ref[...])[:, :3]


def _row_block(blk, width):
    return pl.BlockSpec((blk, width), lambda i: (i, 0))


def _full(shape):
    return pl.BlockSpec(shape, lambda i: tuple(0 for _ in shape))


NBLK = 2000
EBLK = 2560


def _prep(h, wa, wb, be1):
    return pl.pallas_call(
        _prep_body,
        grid=(N // NBLK,),
        in_specs=[
            _row_block(NBLK, HID),
            _full((HID, HID)),
            _full((HID, HID)),
            _full((1, HID)),
        ],
        out_specs=[_row_block(NBLK, HID), _row_block(NBLK, HID)],
        out_shape=[
            jax.ShapeDtypeStruct((N, HID), _f32),
            jax.ShapeDtypeStruct((N, HID), _f32),
        ],
    )(h, wa, wb, be1)


def _edge(s, rel, ea, wc, wd, we2, be2, wc1, bc1, wc2):
    eh = s.shape[0]
    return pl.pallas_call(
        _edge_body,
        grid=(eh // EBLK,),
        in_specs=[
            _row_block(EBLK, HID),
            _row_block(EBLK, PW),
            _row_block(EBLK, EDGE_DIM),
            _full((EDGE_DIM, HID)),
            _full((1, HID)),
            _full((HID, HID)),
            _full((1, HID)),
            _full((HID, HID)),
            _full((1, HID)),
            _full((HID, 1)),
        ],
        out_specs=[_row_block(EBLK, HID), _row_block(EBLK, PW)],
        out_shape=[
            jax.ShapeDtypeStruct((eh, HID), _f32),
            jax.ShapeDtypeStruct((eh, PW), _f32),
        ],
        name="edge_mlp",
    )(s, rel, ea, wc, wd, we2, be2, wc1, bc1, wc2)


def _node(h, m0, m1, p0, p1, pos, wn1a, wn1b, bn1, wn2, bn2):
    return pl.pallas_call(
        _node_body,
        grid=(N // NBLK,),
        in_specs=[
            _row_block(NBLK, HID),
            _row_block(NBLK, HID),
            _row_block(NBLK, HID),
            _row_block(NBLK, PW),
            _row_block(NBLK, PW),
            _row_block(NBLK, 3),
            _full((HID, HID)),
            _full((HID, HID)),
            _full((1, HID)),
            _full((HID, HID)),
            _full((1, HID)),
        ],
        out_specs=[_row_block(NBLK, HID), _row_block(NBLK, 3)],
        out_shape=[
            jax.ShapeDtypeStruct((N, HID), _f32),
            jax.ShapeDtypeStruct((N, 3), _f32),
        ],
    )(h, m0, m1, p0, p1, pos, wn1a, wn1b, bn1, wn2, bn2)


NCHS = (42, 42, 41)


def kernel(h, pos, edge_index, edge_attr,
           We1, be1, We2, be2, Wn1, bn1, Wn2, bn2, Wc1, bc1, Wc2):
    row = edge_index[0].astype(jnp.int32)
    col = edge_index[1].astype(jnp.int32)
    pos16 = jnp.pad(pos, ((0, 0), (0, PW - 3)))

    a_tab, b_tab = _prep(h, We1[:HID], We1[HID:2 * HID],
                         be1.reshape(1, HID))

    wc = We1[2 * HID:2 * HID + EDGE_DIM]
    wd = We1[2 * HID + EDGE_DIM:].reshape(1, HID)
    be2r = be2.reshape(1, HID)
    bc1r = bc1.reshape(1, HID)

    cols, ms, cos = [], [], []
    off = 0
    for nch in NCHS:
        ep = NW * nch * C
        rowp = lax.dynamic_slice_in_dim(row, off, ep).reshape(NW, nch, C)
        colp = lax.dynamic_slice_in_dim(col, off, ep).reshape(NW, nch, C)
        eap = lax.dynamic_slice_in_dim(edge_attr, off, ep)
        sp, relp = _gather(nch)(rowp, colp, a_tab, b_tab, pos16)
        mp, cop = _edge(sp, relp, eap, wc, wd, We2, be2r, Wc1, bc1r, Wc2)
        cols.append(colp)
        ms.append(mp)
        cos.append(cop)
        off += ep

    z128 = jnp.zeros((N, HID), _f32)
    z16 = jnp.zeros((N, PW), _f32)
    m_parts, c_parts = _scatter(NCHS)(*cols, *ms, *cos, z128, z16)

    h_new, pos_new = _node(h, m_parts[0], m_parts[1],
                           c_parts[0], c_parts[1], pos,
                           Wn1[:HID], Wn1[HID:], bn1.reshape(1, HID),
                           Wn2, bn2.reshape(1, HID))
    return h_new, pos_new


# speedup vs baseline: 4.9307x; 1.0181x over previous
import functools

import jax
import jax.numpy as jnp
from jax import lax
from jax.experimental import pallas as pl
from jax.experimental.pallas import tpu as pltpu
from jax.experimental.pallas import tpu_sc as plsc

N = 10000
E = 320000
HID = 128
EDGE_DIM = 16
PW = 16

NC = 2
NS = 16
NW = NC * NS
EW = E // NW
C = 80
NCH = EW // C
RPT = N // NS

_f32 = jnp.float32


@functools.lru_cache(maxsize=None)
def _sc_mesh():
    return plsc.VectorSubcoreMesh(core_axis_name="c", subcore_axis_name="s",
                                  num_cores=NC, num_subcores=NS)


def _make_gather_body(nch):
  ewh = nch * C

  def _gather_body(row_h, col_h, a_h, b_h, p_h,
                   s_out_h, rel_h,
                   idxr, idxc,
                   ba0, bb0, bpr0, bpc0,
                   ba1, bb1, bpr1, bpc1,
                   sg0, sg1, sw0, sw1):
    cid = lax.axis_index("c")
    sid = lax.axis_index("s")
    wid = sid * NC + cid

    pltpu.sync_copy(row_h.at[wid], idxr)
    pltpu.sync_copy(col_h.at[wid], idxc)

    def startg(j, ba, bb, bpr, bpc, sg):
        ia = idxr.at[j]
        ic = idxc.at[j]
        pltpu.async_copy(a_h.at[ia], ba, sg)
        pltpu.async_copy(b_h.at[ic], bb, sg)
        pltpu.async_copy(p_h.at[ia], bpr, sg)
        pltpu.async_copy(p_h.at[ic], bpc, sg)

    def waitg(ba, bb, bpr, bpc, sg):
        pltpu.make_async_copy(a_h.at[pl.ds(0, C)], ba, sg).wait()
        pltpu.make_async_copy(b_h.at[pl.ds(0, C)], bb, sg).wait()
        pltpu.make_async_copy(p_h.at[pl.ds(0, C)], bpr, sg).wait()
        pltpu.make_async_copy(p_h.at[pl.ds(0, C)], bpc, sg).wait()

    def combine(ba, bb, bpr, bpc):
        def vrow(i, c2):
            for k in range(HID // 16):
                sl = (i, pl.ds(k * 16, 16))
                ba[sl] = ba[sl] + bb[sl]
            pp = (i, pl.ds(0, 16))
            bpr[pp] = bpr[pp] - bpc[pp]
            return c2
        lax.fori_loop(0, C, vrow, 0)

    def startw(j, ba, bpr, sw):
        base = wid * ewh + j * C
        pltpu.async_copy(ba, s_out_h.at[pl.ds(base, C)], sw)
        pltpu.async_copy(bpr, rel_h.at[pl.ds(base, C)], sw)

    def waitw(ba, bpr, sw):
        pltpu.make_async_copy(ba, s_out_h.at[pl.ds(0, C)], sw).wait()
        pltpu.make_async_copy(bpr, rel_h.at[pl.ds(0, C)], sw).wait()

    set0 = (ba0, bb0, bpr0, bpc0)
    set1 = (ba1, bb1, bpr1, bpc1)
    startg(0, *set0, sg0)

    def pipe(jj, carry):
        j0 = 2 * jj
        j1 = j0 + 1
        startg(j1, *set1, sg1)
        waitg(*set0, sg0)
        combine(*set0)
        startw(j0, ba0, bpr0, sw0)
        waitg(*set1, sg1)
        combine(*set1)
        startw(j1, ba1, bpr1, sw1)
        waitw(ba0, bpr0, sw0)
        startg(j0 + 2, *set0, sg0)
        waitw(ba1, bpr1, sw1)
        return carry

    lax.fori_loop(0, (nch - 1) // 2, pipe, 0)
    if nch % 2 == 1:
        waitg(*set0, sg0)
        combine(*set0)
        startw(nch - 1, ba0, bpr0, sw0)
        waitw(ba0, bpr0, sw0)
    else:
        startg(nch - 1, *set1, sg1)
        waitg(*set0, sg0)
        combine(*set0)
        startw(nch - 2, ba0, bpr0, sw0)
        waitg(*set1, sg1)
        combine(*set1)
        startw(nch - 1, ba1, bpr1, sw1)
        waitw(ba0, bpr0, sw0)
        waitw(ba1, bpr1, sw1)

  return _gather_body


@functools.lru_cache(maxsize=None)
def _gather(nch):
  eh = NW * nch * C
  return pl.kernel(
    _make_gather_body(nch),
    out_type=(
        jax.ShapeDtypeStruct((eh, HID), _f32),
        jax.ShapeDtypeStruct((eh, PW), _f32),
    ),
    mesh=_sc_mesh(),
    compiler_params=pltpu.CompilerParams(use_tc_tiling_on_sc=False),
    scratch_types=[
        pltpu.VMEM((nch, C), jnp.int32),
        pltpu.VMEM((nch, C), jnp.int32),
        pltpu.VMEM((C, HID), _f32),
        pltpu.VMEM((C, HID), _f32),
        pltpu.VMEM((C, PW), _f32),
        pltpu.VMEM((C, PW), _f32),
        pltpu.VMEM((C, HID), _f32),
        pltpu.VMEM((C, HID), _f32),
        pltpu.VMEM((C, PW), _f32),
        pltpu.VMEM((C, PW), _f32),
        pltpu.SemaphoreType.DMA,
        pltpu.SemaphoreType.DMA,
        pltpu.SemaphoreType.DMA,
        pltpu.SemaphoreType.DMA,
    ],
  )


def _make_scatter_body(nchs):
  np_ = len(nchs)

  def _scatter_body(*refs):
    col_hs = refs[0:np_]
    m_hs = refs[np_:2 * np_]
    c_hs = refs[2 * np_:3 * np_]
    z128_h, z16_h, outm_h, outc_h = refs[3 * np_:3 * np_ + 4]
    (idxc, bm0, bc0, bm1, bc1, m_acc, c_acc,
     sl0, sl1) = refs[3 * np_ + 4:]
    cid = lax.axis_index("c")
    sid = lax.axis_index("s")
    wid = sid * NC + cid

    rows = pl.ds(sid * RPT, RPT)
    pltpu.sync_copy(z128_h.at[rows], m_acc.at[rows])
    pltpu.sync_copy(z16_h.at[rows], c_acc.at[rows])
    off = 0
    for p in range(np_):
        pltpu.sync_copy(col_hs[p].at[wid], idxc.at[pl.ds(off, nchs[p])])
        off += nchs[p]
    plsc.subcore_barrier()

    def run(m_h, c_h, joff, nch):
        ewh = nch * C

        def startl(j, bm, bc, sl):
            base = wid * ewh + j * C
            pltpu.async_copy(m_h.at[pl.ds(base, C)], bm, sl)
            pltpu.async_copy(c_h.at[pl.ds(base, C)], bc, sl)

        def waitl(bm, bc, sl):
            pltpu.make_async_copy(m_h.at[pl.ds(0, C)], bm, sl).wait()
            pltpu.make_async_copy(c_h.at[pl.ds(0, C)], bc, sl).wait()

        def scat(j, bm, bc):
            ic = idxc.at[joff + j]
            pltpu.sync_copy(bm, m_acc.at[ic], add=True)
            pltpu.sync_copy(bc, c_acc.at[ic], add=True)

        startl(0, bm0, bc0, sl0)

        def pipe(jj, carry):
            j0 = 2 * jj
            j1 = j0 + 1
            startl(j1, bm1, bc1, sl1)
            waitl(bm0, bc0, sl0)
            scat(j0, bm0, bc0)
            startl(j0 + 2, bm0, bc0, sl0)
            waitl(bm1, bc1, sl1)
            scat(j1, bm1, bc1)
            return carry

        lax.fori_loop(0, (nch - 1) // 2, pipe, 0)
        if nch % 2 == 1:
            waitl(bm0, bc0, sl0)
            scat(nch - 1, bm0, bc0)
        else:
            startl(nch - 1, bm1, bc1, sl1)
            waitl(bm0, bc0, sl0)
            scat(nch - 2, bm0, bc0)
            waitl(bm1, bc1, sl1)
            scat(nch - 1, bm1, bc1)

    off = 0
    for p in range(np_):
        run(m_hs[p], c_hs[p], off, nchs[p])
        off += nchs[p]

    plsc.subcore_barrier()
    pltpu.sync_copy(m_acc.at[rows], outm_h.at[cid, rows])
    pltpu.sync_copy(c_acc.at[rows], outc_h.at[cid, rows])

  return _scatter_body


@functools.lru_cache(maxsize=None)
def _scatter(nchs):
  return pl.kernel(
    _make_scatter_body(nchs),
    out_type=(
        jax.ShapeDtypeStruct((NC, N, HID), _f32),
        jax.ShapeDtypeStruct((NC, N, PW), _f32),
    ),
    mesh=_sc_mesh(),
    compiler_params=pltpu.CompilerParams(use_tc_tiling_on_sc=False),
    scratch_types=[
        pltpu.VMEM((NCH, C), jnp.int32),
        pltpu.VMEM((C, HID), _f32),
        pltpu.VMEM((C, PW), _f32),
        pltpu.VMEM((C, HID), _f32),
        pltpu.VMEM((C, PW), _f32),
        pltpu.VMEM_SHARED((N, HID), _f32),
        pltpu.VMEM_SHARED((N, PW), _f32),
        pltpu.SemaphoreType.DMA,
        pltpu.SemaphoreType.DMA,
    ],
  )


def _prep_body(h_ref, wa_ref, wb_ref, be1_ref, a_ref, b_ref):
    hv = h_ref[...]
    a_ref[...] = (jnp.dot(hv, wa_ref[...], preferred_element_type=_f32)
                  + be1_ref[...])
    b_ref[...] = jnp.dot(hv, wb_ref[...], preferred_element_type=_f32)


def _edge_body(s_ref, rel_ref, ea_ref,
               wc_ref, wd_ref, we2_ref, be2_ref, wc1_ref, bc1_ref, wc2_ref,
               m_ref, co_ref):
    rel = rel_ref[:, :3]
    dist = jnp.sqrt(jnp.sum(rel * rel, axis=1, keepdims=True))
    t = (s_ref[...]
         + jnp.dot(ea_ref[...], wc_ref[...], preferred_element_type=_f32)
         + dist * wd_ref[...])
    t = jax.nn.silu(t)
    m = jax.nn.silu(jnp.dot(t, we2_ref[...], preferred_element_type=_f32)
                    + be2_ref[...])
    m_ref[...] = m
    cw = jnp.dot(
        jax.nn.silu(jnp.dot(m, wc1_ref[...], preferred_element_type=_f32)
                    + bc1_ref[...]),
        wc2_ref[...], preferred_element_type=_f32)
    co = cw * rel / (dist + 1e-8)
    co_ref[...] = jnp.concatenate(
        [co, jnp.zeros((co.shape[0], PW - 3), co.dtype)], axis=1)


def _node_body(h_ref, m0_ref, m1_ref, p0_ref, p1_ref, pos_ref,
               wn1a_ref, wn1b_ref, bn1_ref, wn2_ref, bn2_ref,
               hn_ref, pn_ref):
    hv = h_ref[...]
    mi = m0_ref[...] + m1_ref[...]
    t = jax.nn.silu(jnp.dot(hv, wn1a_ref[...], preferred_element_type=_f32)
                    + jnp.dot(mi, wn1b_ref[...], preferred_element_type=_f32)
                    + bn1_ref[...])
    hn_ref[...] = hv + jnp.dot(t, wn2_ref[...], preferred_element_type=_f32) \
        + bn2_ref[...]
    pn_ref[...] = pos_ref[...] + (p0_ref[...] + p1_ref[...])[:, :3]


def _row_block(blk, width):
    return pl.BlockSpec((blk, width), lambda i: (i, 0))


def _full(shape):
    return pl.BlockSpec(shape, lambda i: tuple(0 for _ in shape))


NBLK = 2000
EBLK = 2560


def _prep(h, wa, wb, be1):
    return pl.pallas_call(
        _prep_body,
        grid=(N // NBLK,),
        in_specs=[
            _row_block(NBLK, HID),
            _full((HID, HID)),
            _full((HID, HID)),
            _full((1, HID)),
        ],
        out_specs=[_row_block(NBLK, HID), _row_block(NBLK, HID)],
        out_shape=[
            jax.ShapeDtypeStruct((N, HID), _f32),
            jax.ShapeDtypeStruct((N, HID), _f32),
        ],
    )(h, wa, wb, be1)


def _edge(s, rel, ea, wc, wd, we2, be2, wc1, bc1, wc2):
    eh = s.shape[0]
    return pl.pallas_call(
        _edge_body,
        grid=(eh // EBLK,),
        in_specs=[
            _row_block(EBLK, HID),
            _row_block(EBLK, PW),
            _row_block(EBLK, EDGE_DIM),
            _full((EDGE_DIM, HID)),
            _full((1, HID)),
            _full((HID, HID)),
            _full((1, HID)),
            _full((HID, HID)),
            _full((1, HID)),
            _full((HID, 1)),
        ],
        out_specs=[_row_block(EBLK, HID), _row_block(EBLK, PW)],
        out_shape=[
            jax.ShapeDtypeStruct((eh, HID), _f32),
            jax.ShapeDtypeStruct((eh, PW), _f32),
        ],
        name="edge_mlp",
    )(s, rel, ea, wc, wd, we2, be2, wc1, bc1, wc2)


def _node(h, m0, m1, p0, p1, pos, wn1a, wn1b, bn1, wn2, bn2):
    return pl.pallas_call(
        _node_body,
        grid=(N // NBLK,),
        in_specs=[
            _row_block(NBLK, HID),
            _row_block(NBLK, HID),
            _row_block(NBLK, HID),
            _row_block(NBLK, PW),
            _row_block(NBLK, PW),
            _row_block(NBLK, 3),
            _full((HID, HID)),
            _full((HID, HID)),
            _full((1, HID)),
            _full((HID, HID)),
            _full((1, HID)),
        ],
        out_specs=[_row_block(NBLK, HID), _row_block(NBLK, 3)],
        out_shape=[
            jax.ShapeDtypeStruct((N, HID), _f32),
            jax.ShapeDtypeStruct((N, 3), _f32),
        ],
    )(h, m0, m1, p0, p1, pos, wn1a, wn1b, bn1, wn2, bn2)


NCHS = (32, 31, 31, 31)


def kernel(h, pos, edge_index, edge_attr,
           We1, be1, We2, be2, Wn1, bn1, Wn2, bn2, Wc1, bc1, Wc2):
    row = edge_index[0].astype(jnp.int32)
    col = edge_index[1].astype(jnp.int32)
    pos16 = jnp.pad(pos, ((0, 0), (0, PW - 3)))

    a_tab, b_tab = _prep(h, We1[:HID], We1[HID:2 * HID],
                         be1.reshape(1, HID))

    wc = We1[2 * HID:2 * HID + EDGE_DIM]
    wd = We1[2 * HID + EDGE_DIM:].reshape(1, HID)
    be2r = be2.reshape(1, HID)
    bc1r = bc1.reshape(1, HID)

    cols, ms, cos = [], [], []
    off = 0
    for nch in NCHS:
        ep = NW * nch * C
        rowp = lax.dynamic_slice_in_dim(row, off, ep).reshape(NW, nch, C)
        colp = lax.dynamic_slice_in_dim(col, off, ep).reshape(NW, nch, C)
        eap = lax.dynamic_slice_in_dim(edge_attr, off, ep)
        sp, relp = _gather(nch)(rowp, colp, a_tab, b_tab, pos16)
        mp, cop = _edge(sp, relp, eap, wc, wd, We2, be2r, Wc1, bc1r, Wc2)
        cols.append(colp)
        ms.append(mp)
        cos.append(cop)
        off += ep

    z128 = jnp.zeros((N, HID), _f32)
    z16 = jnp.zeros((N, PW), _f32)
    m_parts, c_parts = _scatter(NCHS)(*cols, *ms, *cos, z128, z16)

    h_new, pos_new = _node(h, m_parts[0], m_parts[1],
                           c_parts[0], c_parts[1], pos,
                           Wn1[:HID], Wn1[HID:], bn1.reshape(1, HID),
                           Wn2, bn2.reshape(1, HID))
    return h_new, pos_new
